# baseline (speedup 1.0000x reference)
import functools

import jax
import jax.numpy as jnp
from jax import lax
from jax.experimental import pallas as pl
from jax.experimental.pallas import tpu as pltpu
from jax.experimental.pallas import tpu_sc as plsc

DIM = 128
K = 16
L = 200
NR = 32
NC = 2
NS = 16
NW = NC * NS


def _wid():
    return lax.axis_index("s") * NC + lax.axis_index("c")


def _tr_body(in_ref, a_ref, b_ref):
    tr = jnp.transpose(in_ref[...], (1, 0))
    a_ref[...] = tr[:, :DIM]
    b_ref[...] = tr[:, DIM:]


def _make_tr(npad):
    grid = (npad // 128,)
    return pl.pallas_call(
        _tr_body,
        grid=grid,
        in_specs=[pl.BlockSpec((256, 128), lambda i: (0, i))],
        out_specs=(
            pl.BlockSpec((128, 128), lambda i: (i, 0)),
            pl.BlockSpec((128, 128), lambda i: (i, 0)),
        ),
        out_shape=(
            jax.ShapeDtypeStruct((npad, 128), jnp.int32),
            jax.ShapeDtypeStruct((npad, 128), jnp.int32),
        ),
        name="gekg_rec_tr_tc",
    )


H0 = 128
H1 = L - H0


def _user_emb_body(bpw, users_hbm, recA_hbm, recB_hbm, table_hbm, out_hbm,
                   users_v, recA_v, recB_v, rows_v, out_v,
                   sema, semb, semg0, semg1):
    base = _wid() * bpw
    pltpu.sync_copy(users_hbm.at[pl.ds(base, bpw)], users_v)
    pltpu.async_copy(recA_hbm.at[users_v], recA_v, sema)
    pltpu.async_copy(recB_hbm.at[users_v], recB_v, semb)
    pltpu.make_async_copy(recA_hbm.at[users_v], recA_v, sema).wait()
    pltpu.make_async_copy(recB_hbm.at[users_v], recB_v, semb).wait()

    semg = (semg0, semg1)

    def issue(u, p):
        pltpu.async_copy(table_hbm.at[recA_v.at[u]],
                         rows_v.at[p, pl.ds(0, H0)], semg[p])
        pltpu.async_copy(table_hbm.at[recB_v.at[u, pl.ds(0, H1)]],
                         rows_v.at[p, pl.ds(H0, H1)], semg[p])

    def drain(p):
        pltpu.make_async_copy(table_hbm.at[recA_v.at[0]],
                              rows_v.at[p, pl.ds(0, H0)], semg[p]).wait()
        pltpu.make_async_copy(table_hbm.at[recB_v.at[0, pl.ds(0, H1)]],
                              rows_v.at[p, pl.ds(H0, H1)], semg[p]).wait()

    def accum(u, p):
        def body(r, accs):
            return tuple(accs[j] + rows_v[p, r, pl.ds(16 * j, 16)]
                         for j in range(8))
        accs = lax.fori_loop(
            0, L, body, tuple(jnp.zeros((16,), jnp.float32) for _ in range(8)))
        for j in range(8):
            out_v[u, pl.ds(16 * j, 16)] = accs[j]

    issue(0, 0)

    def outer(t, carry):
        for p in range(2):
            u = 2 * t + p

            @pl.when(u + 1 < bpw)
            def _():
                issue(u + 1, 1 - p)

            drain(p)
            accum(u, p)
        return carry

    lax.fori_loop(0, bpw // 2, outer, 0)
    pltpu.sync_copy(out_v, out_hbm.at[pl.ds(base, bpw)])


def _make_user_emb(b):
    bpw = b // NW
    mesh = plsc.VectorSubcoreMesh(core_axis_name="c", subcore_axis_name="s")
    return pl.kernel(
        functools.partial(_user_emb_body, bpw),
        out_type=jax.ShapeDtypeStruct((b, DIM), jnp.float32),
        mesh=mesh,
        scratch_types=[
            pltpu.VMEM((bpw,), jnp.int32),
            pltpu.VMEM((bpw, 128), jnp.int32),
            pltpu.VMEM((bpw, 128), jnp.int32),
            pltpu.VMEM((2, L, DIM), jnp.float32),
            pltpu.VMEM((bpw, DIM), jnp.float32),
            pltpu.SemaphoreType.DMA,
            pltpu.SemaphoreType.DMA,
            pltpu.SemaphoreType.DMA,
            pltpu.SemaphoreType.DMA,
        ],
        name="gekg_user_emb_sc",
    )


def _nbr_body(bpw, items_hbm, entidx_hbm, table_hbm,
              ent_out, item_out,
              idx_v, nbr_v, item_v, ent_v,
              sem_b, semg0, semg1, semw0, semw1):
    base = _wid() * bpw
    nchunk = bpw * K // 128
    pltpu.sync_copy(items_hbm.at[pl.ds(base, bpw)], idx_v)
    pltpu.async_copy(table_hbm.at[idx_v], item_v, sem_b)
    pltpu.sync_copy(entidx_hbm.at[pl.ds(_wid() * nchunk, nchunk)], nbr_v)
    pltpu.make_async_copy(table_hbm.at[idx_v], item_v, sem_b).wait()
    pltpu.sync_copy(item_v, item_out.at[pl.ds(base, bpw)])

    semg = (semg0, semg1)
    semw = (semw0, semw1)

    def g_issue(c, p):
        pltpu.async_copy(table_hbm.at[nbr_v.at[c]], ent_v.at[p], semg[p])

    def g_wait(p):
        pltpu.make_async_copy(table_hbm.at[nbr_v.at[0]],
                              ent_v.at[p], semg[p]).wait()

    def wback(c, p):
        pltpu.async_copy(ent_v.at[p],
                         ent_out.at[pl.ds(base * K + 128 * c, 128)], semw[p])

    def wb_wait(p):
        pltpu.make_async_copy(ent_v.at[p],
                              ent_out.at[pl.ds(0, 128)], semw[p]).wait()

    g_issue(0, 0)
    for c in range(nchunk):
        p = c % 2
        if c + 1 < nchunk:
            if c >= 1:
                wb_wait(1 - p)
            g_issue(c + 1, 1 - p)
        g_wait(p)
        wback(c, p)
    wb_wait(0)
    wb_wait(1)


def _make_nbr(b):
    bpw = b // NW
    mesh = plsc.VectorSubcoreMesh(core_axis_name="c", subcore_axis_name="s")
    return pl.kernel(
        functools.partial(_nbr_body, bpw),
        out_type=(
            jax.ShapeDtypeStruct((b * K, DIM), jnp.float32),
            jax.ShapeDtypeStruct((b, DIM), jnp.float32),
        ),
        mesh=mesh,
        scratch_types=[
            pltpu.VMEM((bpw,), jnp.int32),
            pltpu.VMEM((bpw * K // 128, 128), jnp.int32),
            pltpu.VMEM((bpw, DIM), jnp.float32),
            pltpu.VMEM((2, 128, DIM), jnp.float32),
            pltpu.SemaphoreType.DMA,
            pltpu.SemaphoreType.DMA,
            pltpu.SemaphoreType.DMA,
            pltpu.SemaphoreType.DMA,
            pltpu.SemaphoreType.DMA,
        ],
        name="gekg_nbr_gather_sc",
    )


def _attn_body(ent_ref, rel_ref, item_ref, user_ref, rtab_ref,
               wge_ref, wgr_ref, wae_ref, war_ref, bg_ref, ba_ref,
               out_ref, gen_ref):
    iota32 = lax.broadcasted_iota(jnp.int32, (1, NR), 1)
    wae = wae_ref[...]
    war = war_ref[...]
    bg = bg_ref[...]
    ba = ba_ref[0, 0]
    rtab = rtab_ref[...]
    wge = wge_ref[...]
    wgr = wgr_ref[...]
    s1l, s2l = [], []
    for k in range(K):
        ent_k = ent_ref[k]
        oh = (rel_ref[:, k:k + 1] == iota32).astype(jnp.float32)
        rel_k = jnp.dot(oh, rtab, preferred_element_type=jnp.float32)
        rs = jnp.sum(rel_k * war, axis=1, keepdims=True)
        s1l.append(jnp.sum(ent_k * wae, axis=1, keepdims=True) + rs + ba)
        gen_k = jax.nn.sigmoid(
            jnp.dot(ent_k, wge, preferred_element_type=jnp.float32)
            + jnp.dot(rel_k, wgr, preferred_element_type=jnp.float32) + bg)
        gen_ref[k] = gen_k
        s2l.append(jnp.sum(gen_k * wae, axis=1, keepdims=True) + rs + ba)
    w1 = jax.nn.sigmoid(jnp.concatenate(s1l, axis=1))
    w2 = jax.nn.sigmoid(jnp.concatenate(s2l, axis=1))
    nw1 = jax.nn.softmax(w1, axis=1)
    nw2 = jax.nn.softmax(w2, axis=1)
    acc = item_ref[...]
    for k in range(K):
        acc = (acc + ent_ref[k] * nw1[:, k:k + 1]
               + gen_ref[k] * nw2[:, k:k + 1])
    out_ref[...] = jax.nn.sigmoid(
        jnp.sum(user_ref[...] * acc, axis=1, keepdims=True))


def _make_attn(b, bt):
    grid = (b // bt,)
    return pl.pallas_call(
        _attn_body,
        grid=grid,
        in_specs=[
            pl.BlockSpec((K, bt, DIM), lambda i: (0, i, 0)),
            pl.BlockSpec((bt, K), lambda i: (i, 0)),
            pl.BlockSpec((bt, DIM), lambda i: (i, 0)),
            pl.BlockSpec((bt, DIM), lambda i: (i, 0)),
            pl.BlockSpec((NR, DIM), lambda i: (0, 0)),
            pl.BlockSpec((DIM, DIM), lambda i: (0, 0)),
            pl.BlockSpec((DIM, DIM), lambda i: (0, 0)),
            pl.BlockSpec((1, DIM), lambda i: (0, 0)),
            pl.BlockSpec((1, DIM), lambda i: (0, 0)),
            pl.BlockSpec((1, DIM), lambda i: (0, 0)),
            pl.BlockSpec((1, 1), lambda i: (0, 0)),
        ],
        out_specs=pl.BlockSpec((bt, 1), lambda i: (i, 0)),
        out_shape=jax.ShapeDtypeStruct((b, 1), jnp.float32),
        scratch_shapes=[pltpu.VMEM((K, bt, DIM), jnp.float32)],
        name="gekg_attn_tc",
    )


def kernel(pairs, neighbor_entities, neighbor_relations, user_records,
           entity_embedding_matrix, relation_embedding_matrix, Wg, bg, Wa, ba):
    b = pairs.shape[0]
    users = pairs[:, 0].astype(jnp.int32)
    items = pairs[:, 1].astype(jnp.int32)

    ent_idx = jnp.take(neighbor_entities.astype(jnp.int32), items, axis=0)
    ent_rows_idx = ent_idx.T.reshape(b * K // 128, 128)
    rel_idx = jnp.take(neighbor_relations.astype(jnp.int32), items, axis=0)

    n = user_records.shape[0]
    npad = (n + 127) // 128 * 128
    recA, recB = _make_tr(npad)(user_records.astype(jnp.int32).T)

    user_emb = _make_user_emb(b)(
        users, recA, recB, entity_embedding_matrix)
    ent_rows, item_rows = _make_nbr(b)(
        items, ent_rows_idx, entity_embedding_matrix)

    ent3d = ent_rows.reshape(K, b, DIM)
    wge = Wg[:DIM]
    wgr = Wg[DIM:]
    wae = Wa[:DIM, 0].reshape(1, DIM)
    war = Wa[DIM:, 0].reshape(1, DIM)
    bg2 = bg.reshape(1, DIM)
    ba2 = ba.reshape(1, 1)

    out = _make_attn(b, 256)(ent3d, rel_idx, item_rows, user_emb,
                             relation_embedding_matrix, wge, wgr,
                             wae, war, bg2, ba2)
    return out.reshape(b)


# speedup vs baseline: 1.7805x; 1.7805x over previous
import functools

import jax
import jax.numpy as jnp
from jax import lax
from jax.experimental import pallas as pl
from jax.experimental.pallas import tpu as pltpu
from jax.experimental.pallas import tpu_sc as plsc

DIM = 128
K = 16
L = 200
NR = 32
NC = 2
NS = 16
NW = NC * NS


def _wid():
    return lax.axis_index("s") * NC + lax.axis_index("c")


TRW = 2048


def _tr_body(in_ref, a_ref, b_ref):
    tr = jnp.transpose(in_ref[...], (1, 0))
    a_ref[...] = tr[:, :DIM]
    b_ref[...] = tr[:, DIM:]


def _make_tr(npad):
    grid = (npad // TRW,)
    return pl.pallas_call(
        _tr_body,
        grid=grid,
        in_specs=[pl.BlockSpec((256, TRW), lambda i: (0, i))],
        out_specs=(
            pl.BlockSpec((TRW, 128), lambda i: (i, 0)),
            pl.BlockSpec((TRW, 128), lambda i: (i, 0)),
        ),
        out_shape=(
            jax.ShapeDtypeStruct((npad, 128), jnp.int32),
            jax.ShapeDtypeStruct((npad, 128), jnp.int32),
        ),
        name="gekg_rec_tr_tc",
    )


H0 = 128
H1 = L - H0


def _user_emb_body(bpw, users_hbm, recA_hbm, recB_hbm, table_hbm, out_hbm,
                   users_v, recA_v, recB_v, rows_v, out_v,
                   sema, semb, semg0, semg1):
    base = _wid() * bpw
    pltpu.sync_copy(users_hbm.at[pl.ds(base, bpw)], users_v)
    pltpu.async_copy(recA_hbm.at[users_v], recA_v, sema)
    pltpu.async_copy(recB_hbm.at[users_v], recB_v, semb)
    pltpu.make_async_copy(recA_hbm.at[users_v], recA_v, sema).wait()
    pltpu.make_async_copy(recB_hbm.at[users_v], recB_v, semb).wait()

    semg = (semg0, semg1)

    def issue(u, p):
        pltpu.async_copy(table_hbm.at[recA_v.at[u]],
                         rows_v.at[p, pl.ds(0, H0)], semg[p])
        pltpu.async_copy(table_hbm.at[recB_v.at[u, pl.ds(0, H1)]],
                         rows_v.at[p, pl.ds(H0, H1)], semg[p])

    def drain(p):
        pltpu.make_async_copy(table_hbm.at[recA_v.at[0]],
                              rows_v.at[p, pl.ds(0, H0)], semg[p]).wait()
        pltpu.make_async_copy(table_hbm.at[recB_v.at[0, pl.ds(0, H1)]],
                              rows_v.at[p, pl.ds(H0, H1)], semg[p]).wait()

    def accum(u, p):
        def body(r, accs):
            return tuple(accs[j] + rows_v[p, r, pl.ds(16 * j, 16)]
                         for j in range(8))
        accs = lax.fori_loop(
            0, L, body, tuple(jnp.zeros((16,), jnp.float32) for _ in range(8)))
        for j in range(8):
            out_v[u, pl.ds(16 * j, 16)] = accs[j]

    issue(0, 0)

    def outer(t, carry):
        for p in range(2):
            u = 2 * t + p

            @pl.when(u + 1 < bpw)
            def _():
                issue(u + 1, 1 - p)

            drain(p)
            accum(u, p)
        return carry

    lax.fori_loop(0, bpw // 2, outer, 0)
    pltpu.sync_copy(out_v, out_hbm.at[pl.ds(base, bpw)])


def _make_user_emb(b):
    bpw = b // NW
    mesh = plsc.VectorSubcoreMesh(core_axis_name="c", subcore_axis_name="s")
    return pl.kernel(
        functools.partial(_user_emb_body, bpw),
        out_type=jax.ShapeDtypeStruct((b, DIM), jnp.float32),
        mesh=mesh,
        scratch_types=[
            pltpu.VMEM((bpw,), jnp.int32),
            pltpu.VMEM((bpw, 128), jnp.int32),
            pltpu.VMEM((bpw, 128), jnp.int32),
            pltpu.VMEM((2, L, DIM), jnp.float32),
            pltpu.VMEM((bpw, DIM), jnp.float32),
            pltpu.SemaphoreType.DMA,
            pltpu.SemaphoreType.DMA,
            pltpu.SemaphoreType.DMA,
            pltpu.SemaphoreType.DMA,
        ],
        name="gekg_user_emb_sc",
    )


def _nbr_body(bpw, items_hbm, entidx_hbm, table_hbm,
              ent_out, item_out,
              idx_v, nbr_v, item_v, ent_v,
              sem_b, semg0, semg1, semw0, semw1):
    base = _wid() * bpw
    nchunk = bpw * K // 128
    pltpu.sync_copy(items_hbm.at[pl.ds(base, bpw)], idx_v)
    pltpu.async_copy(table_hbm.at[idx_v], item_v, sem_b)
    pltpu.sync_copy(entidx_hbm.at[pl.ds(_wid() * nchunk, nchunk)], nbr_v)
    pltpu.make_async_copy(table_hbm.at[idx_v], item_v, sem_b).wait()
    pltpu.sync_copy(item_v, item_out.at[pl.ds(base, bpw)])

    semg = (semg0, semg1)
    semw = (semw0, semw1)

    def g_issue(c, p):
        pltpu.async_copy(table_hbm.at[nbr_v.at[c]], ent_v.at[p], semg[p])

    def g_wait(p):
        pltpu.make_async_copy(table_hbm.at[nbr_v.at[0]],
                              ent_v.at[p], semg[p]).wait()

    def wback(c, p):
        pltpu.async_copy(ent_v.at[p],
                         ent_out.at[pl.ds(base * K + 128 * c, 128)], semw[p])

    def wb_wait(p):
        pltpu.make_async_copy(ent_v.at[p],
                              ent_out.at[pl.ds(0, 128)], semw[p]).wait()

    g_issue(0, 0)
    for c in range(nchunk):
        p = c % 2
        if c + 1 < nchunk:
            if c >= 1:
                wb_wait(1 - p)
            g_issue(c + 1, 1 - p)
        g_wait(p)
        wback(c, p)
    wb_wait(0)
    wb_wait(1)


def _make_nbr(b):
    bpw = b // NW
    mesh = plsc.VectorSubcoreMesh(core_axis_name="c", subcore_axis_name="s")
    return pl.kernel(
        functools.partial(_nbr_body, bpw),
        out_type=(
            jax.ShapeDtypeStruct((b * K, DIM), jnp.float32),
            jax.ShapeDtypeStruct((b, DIM), jnp.float32),
        ),
        mesh=mesh,
        scratch_types=[
            pltpu.VMEM((bpw,), jnp.int32),
            pltpu.VMEM((bpw * K // 128, 128), jnp.int32),
            pltpu.VMEM((bpw, DIM), jnp.float32),
            pltpu.VMEM((2, 128, DIM), jnp.float32),
            pltpu.SemaphoreType.DMA,
            pltpu.SemaphoreType.DMA,
            pltpu.SemaphoreType.DMA,
            pltpu.SemaphoreType.DMA,
            pltpu.SemaphoreType.DMA,
        ],
        name="gekg_nbr_gather_sc",
    )


def _attn_body(ent_ref, rel_ref, item_ref, user_ref, rtab_ref,
               wge_ref, wgr_ref, wae_ref, war_ref, bg_ref, ba_ref,
               out_ref, gen_ref):
    iota32 = lax.broadcasted_iota(jnp.int32, (1, NR), 1)
    wae = wae_ref[...]
    war = war_ref[...]
    bg = bg_ref[...]
    ba = ba_ref[0, 0]
    rtab = rtab_ref[...]
    wge = wge_ref[...]
    wgr = wgr_ref[...]
    s1l, s2l = [], []
    for k in range(K):
        ent_k = ent_ref[k]
        oh = (rel_ref[:, k:k + 1] == iota32).astype(jnp.float32)
        rel_k = jnp.dot(oh, rtab, preferred_element_type=jnp.float32)
        rs = jnp.sum(rel_k * war, axis=1, keepdims=True)
        s1l.append(jnp.sum(ent_k * wae, axis=1, keepdims=True) + rs + ba)
        gen_k = jax.nn.sigmoid(
            jnp.dot(ent_k, wge, preferred_element_type=jnp.float32)
            + jnp.dot(rel_k, wgr, preferred_element_type=jnp.float32) + bg)
        gen_ref[k] = gen_k
        s2l.append(jnp.sum(gen_k * wae, axis=1, keepdims=True) + rs + ba)
    w1 = jax.nn.sigmoid(jnp.concatenate(s1l, axis=1))
    w2 = jax.nn.sigmoid(jnp.concatenate(s2l, axis=1))
    nw1 = jax.nn.softmax(w1, axis=1)
    nw2 = jax.nn.softmax(w2, axis=1)
    acc = item_ref[...]
    for k in range(K):
        acc = (acc + ent_ref[k] * nw1[:, k:k + 1]
               + gen_ref[k] * nw2[:, k:k + 1])
    out_ref[...] = jax.nn.sigmoid(
        jnp.sum(user_ref[...] * acc, axis=1, keepdims=True))


def _make_attn(b, bt):
    grid = (b // bt,)
    return pl.pallas_call(
        _attn_body,
        grid=grid,
        in_specs=[
            pl.BlockSpec((K, bt, DIM), lambda i: (0, i, 0)),
            pl.BlockSpec((bt, K), lambda i: (i, 0)),
            pl.BlockSpec((bt, DIM), lambda i: (i, 0)),
            pl.BlockSpec((bt, DIM), lambda i: (i, 0)),
            pl.BlockSpec((NR, DIM), lambda i: (0, 0)),
            pl.BlockSpec((DIM, DIM), lambda i: (0, 0)),
            pl.BlockSpec((DIM, DIM), lambda i: (0, 0)),
            pl.BlockSpec((1, DIM), lambda i: (0, 0)),
            pl.BlockSpec((1, DIM), lambda i: (0, 0)),
            pl.BlockSpec((1, DIM), lambda i: (0, 0)),
            pl.BlockSpec((1, 1), lambda i: (0, 0)),
        ],
        out_specs=pl.BlockSpec((bt, 1), lambda i: (i, 0)),
        out_shape=jax.ShapeDtypeStruct((b, 1), jnp.float32),
        scratch_shapes=[pltpu.VMEM((K, bt, DIM), jnp.float32)],
        name="gekg_attn_tc",
    )


def kernel(pairs, neighbor_entities, neighbor_relations, user_records,
           entity_embedding_matrix, relation_embedding_matrix, Wg, bg, Wa, ba):
    b = pairs.shape[0]
    users = pairs[:, 0].astype(jnp.int32)
    items = pairs[:, 1].astype(jnp.int32)

    ent_idx = jnp.take(neighbor_entities.astype(jnp.int32), items, axis=0)
    ent_rows_idx = ent_idx.T.reshape(b * K // 128, 128)
    rel_idx = jnp.take(neighbor_relations.astype(jnp.int32), items, axis=0)

    n = user_records.shape[0]
    npad = (n + TRW - 1) // TRW * TRW
    recA, recB = _make_tr(npad)(user_records.astype(jnp.int32).T)

    user_emb = _make_user_emb(b)(
        users, recA, recB, entity_embedding_matrix)
    ent_rows, item_rows = _make_nbr(b)(
        items, ent_rows_idx, entity_embedding_matrix)

    ent3d = ent_rows.reshape(K, b, DIM)
    wge = Wg[:DIM]
    wgr = Wg[DIM:]
    wae = Wa[:DIM, 0].reshape(1, DIM)
    war = Wa[DIM:, 0].reshape(1, DIM)
    bg2 = bg.reshape(1, DIM)
    ba2 = ba.reshape(1, 1)

    out = _make_attn(b, 256)(ent3d, rel_idx, item_rows, user_emb,
                             relation_embedding_matrix, wge, wgr,
                             wae, war, bg2, ba2)
    return out.reshape(b)


# speedup vs baseline: 2.3941x; 1.3447x over previous
import functools

import jax
import jax.numpy as jnp
from jax import lax
from jax.experimental import pallas as pl
from jax.experimental.pallas import tpu as pltpu
from jax.experimental.pallas import tpu_sc as plsc

DIM = 128
K = 16
L = 200
NR = 32
NC = 2
NS = 16
NW = NC * NS


def _wid():
    return lax.axis_index("s") * NC + lax.axis_index("c")


TRW = 2048


def _tr_body(in_ref, a_ref, b_ref):
    tr = jnp.transpose(in_ref[...], (1, 0))
    a_ref[...] = tr[:, :DIM]
    b_ref[...] = tr[:, DIM:]


def _make_tr(npad):
    grid = (npad // TRW,)
    return pl.pallas_call(
        _tr_body,
        grid=grid,
        in_specs=[pl.BlockSpec((256, TRW), lambda i: (0, i))],
        out_specs=(
            pl.BlockSpec((TRW, 128), lambda i: (i, 0)),
            pl.BlockSpec((TRW, 128), lambda i: (i, 0)),
        ),
        out_shape=(
            jax.ShapeDtypeStruct((npad, 128), jnp.int32),
            jax.ShapeDtypeStruct((npad, 128), jnp.int32),
        ),
        name="gekg_rec_tr_tc",
    )


H0 = 128
H1 = L - H0


def _user_emb_body(bpw, users_hbm, recA_hbm, recB_hbm, table_hbm, out_hbm,
                   users_v, recA_v, recB_v, rows_v, out_v,
                   sema, semb, semg0, semg1):
    base = _wid() * bpw
    pltpu.sync_copy(users_hbm.at[pl.ds(base, bpw)], users_v)
    pltpu.async_copy(recA_hbm.at[users_v], recA_v, sema)
    pltpu.async_copy(recB_hbm.at[users_v], recB_v, semb)
    pltpu.make_async_copy(recA_hbm.at[users_v], recA_v, sema).wait()
    pltpu.make_async_copy(recB_hbm.at[users_v], recB_v, semb).wait()

    semg = (semg0, semg1)

    def issue(u, p):
        pltpu.async_copy(table_hbm.at[recA_v.at[u]],
                         rows_v.at[p, pl.ds(0, H0)], semg[p])
        pltpu.async_copy(table_hbm.at[recB_v.at[u, pl.ds(0, H1)]],
                         rows_v.at[p, pl.ds(H0, H1)], semg[p])

    def drain(p):
        pltpu.make_async_copy(table_hbm.at[recA_v.at[0]],
                              rows_v.at[p, pl.ds(0, H0)], semg[p]).wait()
        pltpu.make_async_copy(table_hbm.at[recB_v.at[0, pl.ds(0, H1)]],
                              rows_v.at[p, pl.ds(H0, H1)], semg[p]).wait()

    def accum(u, p):
        def body(r, accs):
            return tuple(accs[j] + rows_v[p, r, pl.ds(16 * j, 16)]
                         for j in range(8))
        accs = lax.fori_loop(
            0, L, body, tuple(jnp.zeros((16,), jnp.float32) for _ in range(8)))
        for j in range(8):
            out_v[u, pl.ds(16 * j, 16)] = accs[j]

    issue(0, 0)

    def outer(t, carry):
        for p in range(2):
            u = 2 * t + p

            @pl.when(u + 1 < bpw)
            def _():
                issue(u + 1, 1 - p)

            drain(p)
            accum(u, p)
        return carry

    lax.fori_loop(0, bpw // 2, outer, 0)
    pltpu.sync_copy(out_v, out_hbm.at[pl.ds(base, bpw)])


def _make_user_emb(b):
    bpw = b // NW
    mesh = plsc.VectorSubcoreMesh(core_axis_name="c", subcore_axis_name="s")
    return pl.kernel(
        functools.partial(_user_emb_body, bpw),
        out_type=jax.ShapeDtypeStruct((b, DIM), jnp.float32),
        mesh=mesh,
        scratch_types=[
            pltpu.VMEM((bpw,), jnp.int32),
            pltpu.VMEM((bpw, 128), jnp.int32),
            pltpu.VMEM((bpw, 128), jnp.int32),
            pltpu.VMEM((2, L, DIM), jnp.float32),
            pltpu.VMEM((bpw, DIM), jnp.float32),
            pltpu.SemaphoreType.DMA,
            pltpu.SemaphoreType.DMA,
            pltpu.SemaphoreType.DMA,
            pltpu.SemaphoreType.DMA,
        ],
        name="gekg_user_emb_sc",
    )


def _nbr_body(bpw, items_hbm, entidx_hbm, table_hbm,
              ent_out, item_out,
              idx_v, nbr_v, item_v, ent_v,
              sem_b, semg0, semg1, semw0, semw1):
    base = _wid() * bpw
    nchunk = bpw * K // 128
    pltpu.sync_copy(items_hbm.at[pl.ds(base, bpw)], idx_v)
    pltpu.async_copy(table_hbm.at[idx_v], item_v, sem_b)
    pltpu.sync_copy(entidx_hbm.at[pl.ds(_wid() * nchunk, nchunk)], nbr_v)
    pltpu.make_async_copy(table_hbm.at[idx_v], item_v, sem_b).wait()
    pltpu.sync_copy(item_v, item_out.at[pl.ds(base, bpw)])

    semg = (semg0, semg1)
    semw = (semw0, semw1)

    def g_issue(c, p):
        pltpu.async_copy(table_hbm.at[nbr_v.at[c]], ent_v.at[p], semg[p])

    def g_wait(p):
        pltpu.make_async_copy(table_hbm.at[nbr_v.at[0]],
                              ent_v.at[p], semg[p]).wait()

    def wback(c, p):
        pltpu.async_copy(ent_v.at[p],
                         ent_out.at[pl.ds(base * K + 128 * c, 128)], semw[p])

    def wb_wait(p):
        pltpu.make_async_copy(ent_v.at[p],
                              ent_out.at[pl.ds(0, 128)], semw[p]).wait()

    g_issue(0, 0)
    for c in range(nchunk):
        p = c % 2
        if c + 1 < nchunk:
            if c >= 1:
                wb_wait(1 - p)
            g_issue(c + 1, 1 - p)
        g_wait(p)
        wback(c, p)
    wb_wait(0)
    wb_wait(1)


def _make_nbr(b):
    bpw = b // NW
    mesh = plsc.VectorSubcoreMesh(core_axis_name="c", subcore_axis_name="s")
    return pl.kernel(
        functools.partial(_nbr_body, bpw),
        out_type=(
            jax.ShapeDtypeStruct((b * K, DIM), jnp.float32),
            jax.ShapeDtypeStruct((b, DIM), jnp.float32),
        ),
        mesh=mesh,
        scratch_types=[
            pltpu.VMEM((bpw,), jnp.int32),
            pltpu.VMEM((bpw * K // 128, 128), jnp.int32),
            pltpu.VMEM((bpw, DIM), jnp.float32),
            pltpu.VMEM((2, 128, DIM), jnp.float32),
            pltpu.SemaphoreType.DMA,
            pltpu.SemaphoreType.DMA,
            pltpu.SemaphoreType.DMA,
            pltpu.SemaphoreType.DMA,
            pltpu.SemaphoreType.DMA,
        ],
        name="gekg_nbr_gather_sc",
    )


def _attn_body(ent_ref, rel_ref, item_ref, rtab_ref,
               wge_ref, wgr_ref, wae_ref, war_ref, bg_ref, ba_ref,
               out_ref, gen_ref):
    iota32 = lax.broadcasted_iota(jnp.int32, (1, NR), 1)
    wae = wae_ref[...]
    war = war_ref[...]
    bg = bg_ref[...]
    ba = ba_ref[0, 0]
    rtab = rtab_ref[...]
    wge = wge_ref[...]
    wgr = wgr_ref[...]
    s1l, s2l = [], []
    for k in range(K):
        ent_k = ent_ref[k]
        oh = (rel_ref[:, k:k + 1] == iota32).astype(jnp.float32)
        rel_k = jnp.dot(oh, rtab, preferred_element_type=jnp.float32)
        rs = jnp.sum(rel_k * war, axis=1, keepdims=True)
        s1l.append(jnp.sum(ent_k * wae, axis=1, keepdims=True) + rs + ba)
        gen_k = jax.nn.sigmoid(
            jnp.dot(ent_k, wge, preferred_element_type=jnp.float32)
            + jnp.dot(rel_k, wgr, preferred_element_type=jnp.float32) + bg)
        gen_ref[k] = gen_k
        s2l.append(jnp.sum(gen_k * wae, axis=1, keepdims=True) + rs + ba)
    w1 = jax.nn.sigmoid(jnp.concatenate(s1l, axis=1))
    w2 = jax.nn.sigmoid(jnp.concatenate(s2l, axis=1))
    nw1 = jax.nn.softmax(w1, axis=1)
    nw2 = jax.nn.softmax(w2, axis=1)
    acc = item_ref[...]
    for k in range(K):
        acc = (acc + ent_ref[k] * nw1[:, k:k + 1]
               + gen_ref[k] * nw2[:, k:k + 1])
    out_ref[...] = acc


def _make_attn(b, bt):
    grid = (b // bt,)
    return pl.pallas_call(
        _attn_body,
        grid=grid,
        in_specs=[
            pl.BlockSpec((K, bt, DIM), lambda i: (0, i, 0)),
            pl.BlockSpec((bt, K), lambda i: (i, 0)),
            pl.BlockSpec((bt, DIM), lambda i: (i, 0)),
            pl.BlockSpec((NR, DIM), lambda i: (0, 0)),
            pl.BlockSpec((DIM, DIM), lambda i: (0, 0)),
            pl.BlockSpec((DIM, DIM), lambda i: (0, 0)),
            pl.BlockSpec((1, DIM), lambda i: (0, 0)),
            pl.BlockSpec((1, DIM), lambda i: (0, 0)),
            pl.BlockSpec((1, DIM), lambda i: (0, 0)),
            pl.BlockSpec((1, 1), lambda i: (0, 0)),
        ],
        out_specs=pl.BlockSpec((bt, DIM), lambda i: (i, 0)),
        out_shape=jax.ShapeDtypeStruct((b, DIM), jnp.float32),
        scratch_shapes=[pltpu.VMEM((K, bt, DIM), jnp.float32)],
        name="gekg_attn_tc",
    )


def _dot_body(user_ref, item_ref, ones_ref, out_ref):
    prod = user_ref[...] * item_ref[...]
    out_ref[...] = jax.nn.sigmoid(
        jnp.dot(prod, ones_ref[...], preferred_element_type=jnp.float32))


def _make_dot(b, bt):
    grid = (b // bt,)
    return pl.pallas_call(
        _dot_body,
        grid=grid,
        in_specs=[
            pl.BlockSpec((bt, DIM), lambda i: (i, 0)),
            pl.BlockSpec((bt, DIM), lambda i: (i, 0)),
            pl.BlockSpec((DIM, 8), lambda i: (0, 0)),
        ],
        out_specs=pl.BlockSpec((bt, 8), lambda i: (i, 0)),
        out_shape=jax.ShapeDtypeStruct((b, 8), jnp.float32),
        name="gekg_dot_tc",
    )


def kernel(pairs, neighbor_entities, neighbor_relations, user_records,
           entity_embedding_matrix, relation_embedding_matrix, Wg, bg, Wa, ba):
    b = pairs.shape[0]
    users = pairs[:, 0].astype(jnp.int32)
    items = pairs[:, 1].astype(jnp.int32)

    ent_idx = jnp.take(neighbor_entities.astype(jnp.int32), items, axis=0)
    ent_rows_idx = ent_idx.T.reshape(b * K // 128, 128)
    rel_idx = jnp.take(neighbor_relations.astype(jnp.int32), items, axis=0)

    n = user_records.shape[0]
    npad = (n + TRW - 1) // TRW * TRW
    recA, recB = _make_tr(npad)(user_records.astype(jnp.int32).T)

    ent_rows, item_rows = _make_nbr(b)(
        items, ent_rows_idx, entity_embedding_matrix)
    user_emb = _make_user_emb(b)(
        users, recA, recB, entity_embedding_matrix)

    ent3d = ent_rows.reshape(K, b, DIM)
    wge = Wg[:DIM]
    wgr = Wg[DIM:]
    wae = Wa[:DIM, 0].reshape(1, DIM)
    war = Wa[DIM:, 0].reshape(1, DIM)
    bg2 = bg.reshape(1, DIM)
    ba2 = ba.reshape(1, 1)

    item_emb = _make_attn(b, 256)(ent3d, rel_idx, item_rows,
                                  relation_embedding_matrix, wge, wgr,
                                  wae, war, bg2, ba2)
    ones = jnp.ones((DIM, 8), jnp.float32)
    out = _make_dot(b, 1024)(user_emb, item_emb, ones)
    return out[:, 0]
